# Initial kernel scaffold; baseline (speedup 1.0000x reference)
import jax
import jax.numpy as jnp
from jax.experimental import pallas as pl


def kernel(input_ids, word_embeddings, gamma, beta):
    raise NotImplementedError("write your pallas kernel here")


# speedup vs baseline: 2.6784x; 2.6784x over previous
import functools

import jax
import jax.numpy as jnp
from jax import lax
from jax.experimental import pallas as pl
from jax.experimental.pallas import tpu as pltpu
from jax.experimental.pallas import tpu_sc as plsc

EPS = 1e-12

NC = 2
NS = 16
NW = NC * NS

CH = 128


HP = 384


def _ln_table_body(tbl_ref, g_ref, b_ref, out_ref):
    x = tbl_ref[...]
    u = jnp.mean(x, axis=-1, keepdims=True)
    s = jnp.mean((x - u) ** 2, axis=-1, keepdims=True)
    y = g_ref[...] * ((x - u) / jnp.sqrt(s + EPS)) + b_ref[...]
    out_ref[...] = jnp.pad(y, ((0, 0), (0, HP - y.shape[1])))


def _normalize_table(word_embeddings, gamma, beta):
    V, H = word_embeddings.shape
    BR = 2000
    grid = V // BR
    g2 = gamma.reshape(1, H)
    b2 = beta.reshape(1, H)
    return pl.pallas_call(
        _ln_table_body,
        grid=(grid,),
        in_specs=[
            pl.BlockSpec((BR, H), lambda i: (i, 0)),
            pl.BlockSpec((1, H), lambda i: (0, 0)),
            pl.BlockSpec((1, H), lambda i: (0, 0)),
        ],
        out_specs=pl.BlockSpec((BR, HP), lambda i: (i, 0)),
        out_shape=jax.ShapeDtypeStruct((V, HP), jnp.float32),
    )(word_embeddings, g2, b2)


def _make_sc_gather(ntok, H, n_chunks):
    mesh = plsc.VectorSubcoreMesh(core_axis_name="c", subcore_axis_name="s")

    @functools.partial(
        pl.kernel,
        mesh=mesh,
        out_type=jax.ShapeDtypeStruct((ntok, HP), jnp.float32),
        scratch_types=[
            pltpu.VMEM((n_chunks, CH), jnp.int32),
            pltpu.VMEM((CH, HP), jnp.float32),
            pltpu.SemaphoreType.DMA,
        ],
    )
    def gather_kernel(tbl_hbm, idx_hbm, out_hbm, idx_v, rows_v, sem):
        wid = lax.axis_index("s") * NC + lax.axis_index("c")
        base = wid * (n_chunks * CH)
        pltpu.sync_copy(idx_hbm.at[wid], idx_v)
        for c in range(n_chunks):
            pltpu.async_copy(tbl_hbm.at[idx_v.at[c]], rows_v, sem).wait()
            pltpu.sync_copy(rows_v, out_hbm.at[pl.ds(base + c * CH, CH)])

    return gather_kernel


def kernel(input_ids, word_embeddings, gamma, beta):
    B, L = input_ids.shape
    V, H = word_embeddings.shape
    ntok = B * L
    n_chunks = ntok // (NW * CH)

    norm_table = _normalize_table(word_embeddings, gamma, beta)
    idx = input_ids.reshape(NW, n_chunks, CH).astype(jnp.int32)
    out_pad = _make_sc_gather(ntok, H, n_chunks)(norm_table, idx)
    return out_pad[:, :H].reshape(B, L, H)
